# Initial kernel scaffold; baseline (speedup 1.0000x reference)
import jax
import jax.numpy as jnp
from jax.experimental import pallas as pl


def kernel(x, edge_index1, edge_index2, batch, w1, b1, w2, b2, gat_W, gat_att_src, gat_att_dst, gat_b, gcn_W, gcn_b, fc1_W, fc1_b, fc2_W, fc2_b, out_W, out_b):
    raise NotImplementedError("write your pallas kernel here")


# speedup vs baseline: 1.0108x; 1.0108x over previous
import functools

import jax
import jax.numpy as jnp
from jax.experimental import pallas as pl

N = 50000
E = 800000
G = 512
NEG_SLOPE = 0.2
BM = 1000


def _relu(v):
    return jnp.maximum(v, 0.0)


def _leaky(v):
    return jnp.where(v > 0, v, NEG_SLOPE * v)


def _tc1_body(x_ref, w1_ref, b1_ref, w2_ref, b2_ref, vsd_ref,
              h_ref, a_ref, pm_ref):
    h0 = _relu(jnp.dot(x_ref[...], w1_ref[...],
                       preferred_element_type=jnp.float32) + b1_ref[...])
    h1 = _relu(jnp.dot(h0, w2_ref[...],
                       preferred_element_type=jnp.float32) + b2_ref[...])
    h_ref[...] = h1
    a = jnp.dot(h1, vsd_ref[...], preferred_element_type=jnp.float32)
    a_ref[...] = a
    pm_ref[...] = jnp.max(a, axis=0, keepdims=True)[None]


def _tc1(x, w1, b1, w2, b2, vsd):
    nb = N // BM
    return pl.pallas_call(
        _tc1_body,
        grid=(nb,),
        in_specs=[
            pl.BlockSpec((BM, 53), lambda i: (i, 0)),
            pl.BlockSpec((53, 128), lambda i: (0, 0)),
            pl.BlockSpec((1, 128), lambda i: (0, 0)),
            pl.BlockSpec((128, 64), lambda i: (0, 0)),
            pl.BlockSpec((1, 64), lambda i: (0, 0)),
            pl.BlockSpec((64, 2), lambda i: (0, 0)),
        ],
        out_specs=[
            pl.BlockSpec((BM, 64), lambda i: (i, 0)),
            pl.BlockSpec((BM, 2), lambda i: (i, 0)),
            pl.BlockSpec((1, 1, 2), lambda i: (i, 0, 0)),
        ],
        out_shape=[
            jax.ShapeDtypeStruct((N, 64), jnp.float32),
            jax.ShapeDtypeStruct((N, 2), jnp.float32),
            jax.ShapeDtypeStruct((nb, 1, 2), jnp.float32),
        ],
    )(x, w1, b1.reshape(1, 128), w2, b2.reshape(1, 64), vsd)


def _mm_body(x_ref, w_ref, b_ref, o_ref, *, act):
    h = jnp.dot(x_ref[...], w_ref[...],
                preferred_element_type=jnp.float32) + b_ref[...]
    if act:
        h = _relu(h)
    o_ref[...] = h


def _mm(x, w, b, act, bm=BM):
    m, k = x.shape
    n = w.shape[1]
    return pl.pallas_call(
        functools.partial(_mm_body, act=act),
        grid=(m // bm,),
        in_specs=[
            pl.BlockSpec((bm, k), lambda i: (i, 0)),
            pl.BlockSpec((k, n), lambda i: (0, 0)),
            pl.BlockSpec((1, n), lambda i: (0, 0)),
        ],
        out_specs=pl.BlockSpec((bm, n), lambda i: (i, 0)),
        out_shape=jax.ShapeDtypeStruct((m, n), jnp.float32),
    )(x, w, b.reshape(1, n))


def _fin_body(g_ref, w1_ref, b1_ref, w2_ref, b2_ref, w3_ref, b3_ref, o_ref):
    h = _relu(jnp.dot(g_ref[...], w1_ref[...],
                      preferred_element_type=jnp.float32) + b1_ref[...])
    h = _relu(jnp.dot(h, w2_ref[...],
                      preferred_element_type=jnp.float32) + b2_ref[...])
    o_ref[...] = jnp.dot(h, w3_ref[...],
                         preferred_element_type=jnp.float32) + b3_ref[...]


def _final_mlp(g, fc1_W, fc1_b, fc2_W, fc2_b, out_W, out_b):
    return pl.pallas_call(
        _fin_body,
        out_shape=jax.ShapeDtypeStruct((G, 1), jnp.float32),
    )(g, fc1_W, fc1_b.reshape(1, -1), fc2_W, fc2_b.reshape(1, -1),
      out_W, out_b.reshape(1, -1))


def kernel(x, edge_index1, edge_index2, batch, w1, b1, w2, b2, gat_W,
           gat_att_src, gat_att_dst, gat_b, gcn_W, gcn_b, fc1_W, fc1_b,
           fc2_W, fc2_b, out_W, out_b):
    vsd = jnp.stack([gat_W @ gat_att_src, gat_W @ gat_att_dst], axis=1)

    h1, a, pmax = _tc1(x, w1, b1, w2, b2, vsd)
    a_src = a[:, 0]
    a_dst = a[:, 1]
    gmax = jnp.max(pmax[:, 0, 0])

    loop = jnp.arange(N, dtype=edge_index1.dtype)
    src = jnp.concatenate([edge_index1[0], loop])
    dst = jnp.concatenate([edge_index1[1], loop])

    e = _leaky(a_src[src] + a_dst[dst])
    m_node = _leaky(gmax + a_dst)
    ex = jnp.exp(e - m_node[dst])
    den = jax.ops.segment_sum(ex, dst, num_segments=N)
    deg = jax.ops.segment_sum(jnp.ones(src.shape[0], jnp.float32), dst,
                              num_segments=N)
    alpha = ex / (den[dst] + 1e-16)
    agg = jax.ops.segment_sum(alpha[:, None] * h1[src], dst, num_segments=N)

    h2 = _mm(agg, gat_W, gat_b, act=True)

    dinv = jnp.where(deg > 0, jax.lax.rsqrt(jnp.maximum(deg, 1e-12)), 0.0)
    norm = dinv[src] * dinv[dst]
    agg2 = jax.ops.segment_sum(norm[:, None] * h2[src], dst, num_segments=N)

    h3 = _mm(agg2, gcn_W, gcn_b, act=True)

    g = jax.ops.segment_max(h3, batch, num_segments=G)

    return _final_mlp(g, fc1_W, fc1_b, fc2_W, fc2_b, out_W, out_b)


# speedup vs baseline: 19.4692x; 19.2605x over previous
import functools

import jax
import jax.numpy as jnp
from jax import lax
from jax.experimental import pallas as pl
from jax.experimental.pallas import tpu as pltpu
from jax.experimental.pallas import tpu_sc as plsc

N = 50000
E = 800000
G = 512
NEG_SLOPE = 0.2
BM = 1000

NC = 2
NS = 16
NW = NC * NS
CH = 8
SLAB = CH * 128
SL = 26
EPT = SL * SLAB
E_PAD = EPT * NW
N_ACC = 50176
STRIPE = N_ACC // NS


def _relu(v):
    return jnp.maximum(v, 0.0)


def _leaky(v):
    return jnp.where(v > 0, v, NEG_SLOPE * v)


def _tc1_body(x_ref, w1_ref, b1_ref, w2_ref, b2_ref, gw_ref, as_ref, ad_ref,
              hw_ref, a_ref, pm_ref):
    h0 = _relu(jnp.dot(x_ref[...], w1_ref[...],
                       preferred_element_type=jnp.float32) + b1_ref[...])
    h1 = _relu(jnp.dot(h0, w2_ref[...],
                       preferred_element_type=jnp.float32) + b2_ref[...])
    hw = jnp.dot(h1, gw_ref[...], preferred_element_type=jnp.float32)
    hw_ref[...] = hw
    a = jnp.stack([jnp.sum(hw * as_ref[...], axis=1),
                   jnp.sum(hw * ad_ref[...], axis=1)], axis=1)
    a_ref[...] = a
    pm_ref[...] = jnp.max(a, axis=0, keepdims=True)[None]


def _tc1(x, w1, b1, w2, b2, gat_W, att_s, att_d):
    nb = N // BM
    return pl.pallas_call(
        _tc1_body,
        grid=(nb,),
        in_specs=[
            pl.BlockSpec((BM, 53), lambda i: (i, 0)),
            pl.BlockSpec((53, 128), lambda i: (0, 0)),
            pl.BlockSpec((1, 128), lambda i: (0, 0)),
            pl.BlockSpec((128, 64), lambda i: (0, 0)),
            pl.BlockSpec((1, 64), lambda i: (0, 0)),
            pl.BlockSpec((64, 64), lambda i: (0, 0)),
            pl.BlockSpec((1, 64), lambda i: (0, 0)),
            pl.BlockSpec((1, 64), lambda i: (0, 0)),
        ],
        out_specs=[
            pl.BlockSpec((BM, 64), lambda i: (i, 0)),
            pl.BlockSpec((BM, 2), lambda i: (i, 0)),
            pl.BlockSpec((1, 1, 2), lambda i: (i, 0, 0)),
        ],
        out_shape=[
            jax.ShapeDtypeStruct((N, 64), jnp.float32),
            jax.ShapeDtypeStruct((N, 2), jnp.float32),
            jax.ShapeDtypeStruct((nb, 1, 2), jnp.float32),
        ],
    )(x, w1, b1.reshape(1, 128), w2, b2.reshape(1, 64), gat_W,
      att_s.reshape(1, 64), att_d.reshape(1, 64))


def _mm_body(x_ref, w_ref, b_ref, o_ref, *, act):
    h = jnp.dot(x_ref[...], w_ref[...],
                preferred_element_type=jnp.float32) + b_ref[...]
    if act:
        h = _relu(h)
    o_ref[...] = h


def _mm(x, w, b, act, bm=BM):
    m, k = x.shape
    n = w.shape[1]
    return pl.pallas_call(
        functools.partial(_mm_body, act=act),
        grid=(m // bm,),
        in_specs=[
            pl.BlockSpec((bm, k), lambda i: (i, 0)),
            pl.BlockSpec((k, n), lambda i: (0, 0)),
            pl.BlockSpec((1, n), lambda i: (0, 0)),
        ],
        out_specs=pl.BlockSpec((bm, n), lambda i: (i, 0)),
        out_shape=jax.ShapeDtypeStruct((m, n), jnp.float32),
    )(x, w, b.reshape(1, n))


def _fin_body(g_ref, w1_ref, b1_ref, w2_ref, b2_ref, w3_ref, b3_ref, o_ref):
    h = _relu(jnp.dot(g_ref[...], w1_ref[...],
                      preferred_element_type=jnp.float32) + b1_ref[...])
    h = _relu(jnp.dot(h, w2_ref[...],
                      preferred_element_type=jnp.float32) + b2_ref[...])
    o_ref[...] = jnp.dot(h, w3_ref[...],
                         preferred_element_type=jnp.float32) + b3_ref[...]


def _final_mlp(g, fc1_W, fc1_b, fc2_W, fc2_b, out_W, out_b):
    return pl.pallas_call(
        _fin_body,
        out_shape=jax.ShapeDtypeStruct((G, 1), jnp.float32),
    )(g, fc1_W, fc1_b.reshape(1, -1), fc2_W, fc2_b.reshape(1, -1),
      out_W, out_b.reshape(1, -1))


def _sca_body(src_hbm, dst_hbm, dst2_hbm, as_hbm, ad_hbm, m_hbm, zeros_hbm,
              ex_hbm, den_hbm, deg_hbm,
              src_v, dst_v, dst2_v, gs_v, gd_v, gm_v, ex_v, ones_v,
              den_sh, deg_sh, sem):
    c = lax.axis_index("c")
    s = lax.axis_index("s")
    wid = s * NC + c

    @pl.when(s == 0)
    def _():
        pltpu.sync_copy(zeros_hbm, den_sh)
        pltpu.sync_copy(zeros_hbm, deg_sh)

    for k in range(128 // 16):
        ones_v[pl.ds(k * 16, 16)] = jnp.ones((16,), jnp.float32)
    plsc.subcore_barrier()

    base = wid * EPT

    def slab(i, carry):
        sb = base + i * SLAB
        pltpu.sync_copy(src_hbm.at[pl.ds(sb, SLAB)], src_v)
        pltpu.sync_copy(dst_hbm.at[pl.ds(sb, SLAB)], dst_v)
        row0 = pl.multiple_of(sb // 128, 8)
        pltpu.sync_copy(dst2_hbm.at[pl.ds(row0, CH)], dst2_v)
        pltpu.async_copy(as_hbm.at[src_v], gs_v, sem).wait()
        pltpu.async_copy(ad_hbm.at[dst_v], gd_v, sem).wait()
        pltpu.async_copy(m_hbm.at[dst_v], gm_v, sem).wait()

        def vec(k, _):
            sl = pl.ds(k * 16, 16)
            e = gs_v[sl] + gd_v[sl]
            e = jnp.where(e > 0, e, NEG_SLOPE * e)
            ex_v[sl] = jnp.exp(e - gm_v[sl])
            return 0

        lax.fori_loop(0, SLAB // 16, vec, 0)
        pltpu.sync_copy(ex_v, ex_hbm.at[pl.ds(sb, SLAB)])
        for j in range(CH):
            idx = dst2_v.at[j]
            pltpu.sync_copy(ex_v.at[pl.ds(j * 128, 128)],
                            den_sh.at[idx], add=True)
            pltpu.sync_copy(ones_v, deg_sh.at[idx], add=True)
        return carry

    lax.fori_loop(0, SL, slab, 0)
    plsc.subcore_barrier()

    @pl.when(s == 0)
    def _():
        pltpu.sync_copy(den_sh, den_hbm.at[c])
        pltpu.sync_copy(deg_sh, deg_hbm.at[c])


def _sc_edge_scalars(src_all, dst_all, dst2, as_t, ad_t, m_t, zeros_acc):
    mesh = plsc.VectorSubcoreMesh(core_axis_name="c", subcore_axis_name="s")
    f = pl.kernel(
        _sca_body,
        out_type=[
            jax.ShapeDtypeStruct((E_PAD,), jnp.float32),
            jax.ShapeDtypeStruct((NC, N_ACC), jnp.float32),
            jax.ShapeDtypeStruct((NC, N_ACC), jnp.float32),
        ],
        mesh=mesh,
        scratch_types=[
            pltpu.VMEM((SLAB,), jnp.int32),
            pltpu.VMEM((SLAB,), jnp.int32),
            pltpu.VMEM((CH, 128), jnp.int32),
            pltpu.VMEM((SLAB,), jnp.float32),
            pltpu.VMEM((SLAB,), jnp.float32),
            pltpu.VMEM((SLAB,), jnp.float32),
            pltpu.VMEM((SLAB,), jnp.float32),
            pltpu.VMEM((128,), jnp.float32),
            pltpu.VMEM_SHARED((N_ACC,), jnp.float32),
            pltpu.VMEM_SHARED((N_ACC,), jnp.float32),
            pltpu.SemaphoreType.DMA,
        ],
    )
    return f(src_all, dst_all, dst2, as_t, ad_t, m_t, zeros_acc)


ACH = 4
ASLAB = ACH * 128
EPS = E_PAD // NS
ASL = EPS // ASLAB


def _agg_body(scale, *refs):
    if scale:
        (src_hbm, src2_hbm, idx3_hbm, ex_hbm, h_hbm, out_hbm,
         src_v, dst2_v, ex_v, rows_v, acc_sh, sem) = refs
    else:
        (src_hbm, src2_hbm, idx3_hbm, h_hbm, out_hbm,
         src_v, dst2_v, rows_v, acc_sh, sem) = refs
    c = lax.axis_index("c")
    s = lax.axis_index("s")

    z = jnp.zeros((16,), jnp.float32)

    def zrow(i, _):
        rows_v[i, pl.ds(0, 16)] = z
        rows_v[i, pl.ds(16, 16)] = z
        return 0

    lax.fori_loop(0, ASLAB, zrow, 0)
    for t in range(STRIPE // ASLAB):
        pltpu.sync_copy(rows_v,
                        acc_sh.at[pl.ds(s * STRIPE + t * ASLAB, ASLAB)])
    rem = STRIPE % ASLAB
    if rem:
        pltpu.sync_copy(rows_v.at[pl.ds(0, rem)],
                        acc_sh.at[pl.ds(s * STRIPE + STRIPE - rem, rem)])
    plsc.subcore_barrier()

    base = s * EPS

    def slab(i, carry):
        sb = base + i * ASLAB

        @pl.when(c == 0)
        def _():
            pltpu.sync_copy(src_hbm.at[pl.ds(sb, ASLAB)], src_v)

        @pl.when(c == 1)
        def _():
            pltpu.sync_copy(src2_hbm.at[pl.ds(sb, ASLAB)], src_v)

        pltpu.sync_copy(idx3_hbm.at[s * ASL + i], dst2_v)
        if scale:
            pltpu.sync_copy(ex_hbm.at[pl.ds(sb, ASLAB)], ex_v)
        pltpu.async_copy(h_hbm.at[src_v], rows_v, sem).wait()
        if scale:
            def sgrp(k, _):
                w16 = ex_v[pl.ds(k * 16, 16)]
                for j in range(16):
                    e = k * 16 + j
                    w = jnp.full((16,), w16[j], jnp.float32)
                    rows_v[e, pl.ds(0, 16)] = rows_v[e, pl.ds(0, 16)] * w
                    rows_v[e, pl.ds(16, 16)] = rows_v[e, pl.ds(16, 16)] * w
                return 0

            lax.fori_loop(0, ASLAB // 16, sgrp, 0)
        for j in range(ACH):
            pltpu.sync_copy(rows_v.at[pl.ds(j * 128, 128)],
                            acc_sh.at[dst2_v.at[j]], add=True)
        return carry

    lax.fori_loop(0, ASL, slab, 0)
    plsc.subcore_barrier()
    pltpu.sync_copy(acc_sh.at[pl.ds(s * STRIPE, STRIPE)],
                    out_hbm.at[c, pl.ds(s * STRIPE, STRIPE)])


def _sc_aggregate(src_all, src2_all, idx3, h_all, ex=None):
    scale = ex is not None
    mesh = plsc.VectorSubcoreMesh(core_axis_name="c", subcore_axis_name="s")
    scratch = [
        pltpu.VMEM((ASLAB,), jnp.int32),
        pltpu.VMEM((ACH, 128), jnp.int32),
    ]
    if scale:
        scratch.append(pltpu.VMEM((ASLAB,), jnp.float32))
    scratch += [
        pltpu.VMEM((ASLAB, 32), jnp.float32),
        pltpu.VMEM_SHARED((N_ACC, 32), jnp.float32),
        pltpu.SemaphoreType.DMA,
    ]
    f = pl.kernel(
        functools.partial(_agg_body, scale),
        out_type=jax.ShapeDtypeStruct((NC, N_ACC, 32), jnp.float32),
        mesh=mesh,
        scratch_types=scratch,
        compiler_params=pltpu.CompilerParams(use_tc_tiling_on_sc=False),
    )
    args = (src_all, src2_all, idx3) + ((ex,) if scale else ()) + (h_all,)
    out = f(*args)
    return jnp.concatenate([out[0, :N], out[1, :N]], axis=1)


def _split_table(h):
    pad = jnp.zeros((N_ACC - N, 32), jnp.float32)
    return jnp.concatenate([h[:, :32], pad, h[:, 32:], pad], axis=0)


RW = N_ACC // NW
PCH = 224


def _pool_body(h_hbm, b_hbm, out_hbm, bat_v, buf_v, acc_v, sem):
    c = lax.axis_index("c")
    s = lax.axis_index("s")
    wid = s * NC + c
    base = wid * RW

    ninf = jnp.full((16,), -jnp.inf, jnp.float32)

    def initg(g, _):
        for k in range(8):
            acc_v[g, pl.ds(k * 16, 16)] = ninf
        return 0

    lax.fori_loop(0, G + 1, initg, 0)
    pltpu.sync_copy(b_hbm.at[pl.ds(base, RW)], bat_v)

    def chunk(t, _):
        r0 = base + t * PCH
        pltpu.sync_copy(h_hbm.at[pl.ds(r0, PCH)], buf_v)

        def row16(g, _):
            b16 = bat_v[pl.ds(t * PCH + g * 16, 16)]
            for j in range(16):
                b = b16[j]
                r = g * 16 + j
                for k in range(8):
                    sl = pl.ds(k * 16, 16)
                    acc_v[b, sl] = jnp.maximum(acc_v[b, sl], buf_v[r, sl])
            return 0

        lax.fori_loop(0, PCH // 16, row16, 0)
        return 0

    lax.fori_loop(0, RW // PCH, chunk, 0)
    pltpu.sync_copy(acc_v, out_hbm.at[wid])


def _sc_pool(h3p, batp):
    mesh = plsc.VectorSubcoreMesh(core_axis_name="c", subcore_axis_name="s")
    f = pl.kernel(
        _pool_body,
        out_type=jax.ShapeDtypeStruct((NW, G + 1, 128), jnp.float32),
        mesh=mesh,
        scratch_types=[
            pltpu.VMEM((RW,), jnp.int32),
            pltpu.VMEM((PCH, 128), jnp.float32),
            pltpu.VMEM((G + 1, 128), jnp.float32),
            pltpu.SemaphoreType.DMA,
        ],
    )
    return f(h3p, batp)


def _fin2_body(p_ref, w1_ref, b1_ref, w2_ref, b2_ref, w3_ref, b3_ref, o_ref):
    g = jnp.max(p_ref[...], axis=0)[:G]
    h = _relu(jnp.dot(g, w1_ref[...],
                      preferred_element_type=jnp.float32) + b1_ref[...])
    h = _relu(jnp.dot(h, w2_ref[...],
                      preferred_element_type=jnp.float32) + b2_ref[...])
    o_ref[...] = jnp.dot(h, w3_ref[...],
                         preferred_element_type=jnp.float32) + b3_ref[...]


def _final_mlp2(parts, fc1_W, fc1_b, fc2_W, fc2_b, out_W, out_b):
    return pl.pallas_call(
        _fin2_body,
        out_shape=jax.ShapeDtypeStruct((G, 1), jnp.float32),
    )(parts, fc1_W, fc1_b.reshape(1, -1), fc2_W, fc2_b.reshape(1, -1),
      out_W, out_b.reshape(1, -1))


def _pg_body(agg_ref, iden_ref, dinv_ref, gb_ref, gw_ref, o_ref):
    h2 = _relu(agg_ref[...] * iden_ref[...] + gb_ref[...])
    hw2 = jnp.dot(h2, gw_ref[...], preferred_element_type=jnp.float32)
    o_ref[...] = hw2 * dinv_ref[...]


def _post_gat(agg, inv_den, dinv, gat_b, gcn_W):
    return pl.pallas_call(
        _pg_body,
        grid=(N // BM,),
        in_specs=[
            pl.BlockSpec((BM, 64), lambda i: (i, 0)),
            pl.BlockSpec((BM, 1), lambda i: (i, 0)),
            pl.BlockSpec((BM, 1), lambda i: (i, 0)),
            pl.BlockSpec((1, 64), lambda i: (0, 0)),
            pl.BlockSpec((64, 128), lambda i: (0, 0)),
        ],
        out_specs=pl.BlockSpec((BM, 128), lambda i: (i, 0)),
        out_shape=jax.ShapeDtypeStruct((N, 128), jnp.float32),
    )(agg, inv_den.reshape(N, 1), dinv.reshape(N, 1),
      gat_b.reshape(1, 64), gcn_W)


def _pc_body(agg_ref, dinv_ref, b_ref, o_ref):
    o_ref[...] = _relu(agg_ref[...] * dinv_ref[...] + b_ref[...])


def _post_gcn(agg2, dinv, gcn_b):
    return pl.pallas_call(
        _pc_body,
        grid=(N // BM,),
        in_specs=[
            pl.BlockSpec((BM, 128), lambda i: (i, 0)),
            pl.BlockSpec((BM, 1), lambda i: (i, 0)),
            pl.BlockSpec((1, 128), lambda i: (0, 0)),
        ],
        out_specs=pl.BlockSpec((BM, 128), lambda i: (i, 0)),
        out_shape=jax.ShapeDtypeStruct((N, 128), jnp.float32),
    )(agg2, dinv.reshape(N, 1), gcn_b.reshape(1, 128))


def kernel(x, edge_index1, edge_index2, batch, w1, b1, w2, b2, gat_W,
           gat_att_src, gat_att_dst, gat_b, gcn_W, gcn_b, fc1_W, fc1_b,
           fc2_W, fc2_b, out_W, out_b):
    hw, a, pmax = _tc1(x, w1, b1, w2, b2, gat_W, gat_att_src, gat_att_dst)
    a_src = a[:, 0]
    a_dst = a[:, 1]
    gmax = jnp.max(pmax[:, 0, 0])

    loop = jnp.arange(N, dtype=edge_index1.dtype)
    npad = E_PAD - (E + N)
    pad = jnp.full((npad,), N, dtype=edge_index1.dtype)
    src_all = jnp.concatenate([edge_index1[0], loop, pad])
    dst_all = jnp.concatenate([edge_index1[1], loop, pad])
    dst2 = jnp.pad(dst_all, (0, 1024)).reshape(E_PAD // 128 + 8, 128)
    idx3 = jnp.pad(dst_all, (0, 512)).reshape(E_PAD // ASLAB + 1, ACH, 128)

    m_node = _leaky(gmax + a_dst)
    tpad = jnp.zeros((N_ACC - N,), jnp.float32)
    as_t = jnp.concatenate([a_src, tpad])
    ad_t = jnp.concatenate([a_dst, tpad])
    m_t = jnp.concatenate([m_node, tpad])
    zeros_acc = jnp.zeros((N_ACC,), jnp.float32)
    ex_pad, den2, deg2 = _sc_edge_scalars(src_all, dst_all, dst2, as_t, ad_t,
                                          m_t, zeros_acc)
    den = (den2[0] + den2[1])[:N]
    deg = (deg2[0] + deg2[1])[:N]
    inv_den = 1.0 / (den + 1e-16)
    dinv = jnp.where(deg > 0, jax.lax.rsqrt(jnp.maximum(deg, 1e-12)), 0.0)

    src2_all = src_all + N_ACC
    agg = _sc_aggregate(src_all, src2_all, idx3, _split_table(hw), ex=ex_pad)
    hw2s = _post_gat(agg, inv_den, dinv, gat_b, gcn_W)

    agg2a = _sc_aggregate(src_all, src2_all, idx3,
                          _split_table(hw2s[:, :64]))
    agg2b = _sc_aggregate(src_all, src2_all, idx3,
                          _split_table(hw2s[:, 64:]))
    agg2 = jnp.concatenate([agg2a, agg2b], axis=1)
    h3 = _post_gcn(agg2, dinv, gcn_b)

    h3p = jnp.concatenate(
        [h3, jnp.full((N_ACC - N, 128), -jnp.inf, jnp.float32)], axis=0)
    batp = jnp.concatenate([batch, jnp.full((N_ACC - N,), G, batch.dtype)])
    parts = _sc_pool(h3p, batp)

    return _final_mlp2(parts, fc1_W, fc1_b, fc2_W, fc2_b, out_W, out_b)
